# baseline (speedup 1.0000x reference)
import functools

import jax
import jax.numpy as jnp
from jax import lax
from jax.experimental import pallas as pl
from jax.experimental.pallas import tpu as pltpu
from jax.experimental.pallas import tpu_sc as plsc

NN = 10000
EE = 320000
DD = 128
GG = 16

NC = 2
NS = 16
NW = NC * NS
L = 16

NLOC = 313
NPAD = NW * NLOC
EC = 8192
NCHUNK = 40
EPAD = NCHUNK * EC
NBUF = 5
GL = 64


def _matmul_body(x_ref, wc_ref, a_ref, bm_ref):
    p = jnp.dot(x_ref[...], wc_ref[...],
                preferred_element_type=jnp.float32,
                precision=jax.lax.Precision.HIGHEST)
    a_ref[...] = p[:, :DD]
    bm_ref[...] = p[:, DD:]


def _node_matmul(x, wc):
    bn = 2000
    return pl.pallas_call(
        _matmul_body,
        grid=(NN // bn,),
        in_specs=[
            pl.BlockSpec((bn, DD), lambda i: (i, 0)),
            pl.BlockSpec((DD, 2 * DD), lambda i: (0, 0)),
        ],
        out_specs=[
            pl.BlockSpec((bn, DD), lambda i: (i, 0)),
            pl.BlockSpec((bn, DD), lambda i: (i, 0)),
        ],
        out_shape=[
            jax.ShapeDtypeStruct((NN, DD), jnp.float32),
            jax.ShapeDtypeStruct((NN, DD), jnp.float32),
        ],
    )(x, wc)


def _segmax_body(edge_hbm, bm_hbm, out_hbm,
                 ebuf, csrc, cdst, acc, ring, esems, rsems):
    wid = lax.axis_index("s") * NC + lax.axis_index("c")
    base = wid * NLOC
    lower = base
    upper = base + NLOC
    iota = lax.iota(jnp.int32, L)
    neg = jnp.full((L,), -jnp.inf, dtype=jnp.float32)

    def init_body(i, carry):
        for k in range(8):
            acc[pl.ds(i * 8 * L + k * L, L)] = neg
        return carry

    lax.fori_loop(0, (NLOC + 1) * DD // (8 * L), init_body, 0)

    def fire_chunk(c, b):
        pltpu.async_copy(edge_hbm.at[:, pl.ds(c * EC, EC)], ebuf.at[b],
                         esems.at[b])

    def wait_chunk(b):
        pltpu.make_async_copy(edge_hbm.at[:, pl.ds(0, EC)], ebuf.at[b],
                              esems.at[b]).wait()

    def process_chunk(b):
        def scan_body(i, cnt):
            da = ebuf[b, 1, pl.ds(2 * i * L, L)]
            sa = ebuf[b, 0, pl.ds(2 * i * L, L)]
            db = ebuf[b, 1, pl.ds((2 * i + 1) * L, L)]
            sb = ebuf[b, 0, pl.ds((2 * i + 1) * L, L)]
            ma = (da >= lower) & (da < upper)
            mb = (db >= lower) & (db < upper)
            pca = plsc.all_reduce_population_count(ma)[0]
            pcb = plsc.all_reduce_population_count(mb)[0]
            plsc.store_compressed(csrc.at[pl.ds(cnt, L)], sa, mask=ma)
            plsc.store_compressed(cdst.at[pl.ds(cnt, L)], da - lower,
                                  mask=ma)
            cnt2 = cnt + pca
            plsc.store_compressed(csrc.at[pl.ds(cnt2, L)], sb, mask=mb)
            plsc.store_compressed(cdst.at[pl.ds(cnt2, L)], db - lower,
                                  mask=mb)
            return cnt2 + pcb

        cnt = lax.fori_loop(0, EC // L // 2, scan_body, jnp.int32(0))

        for t in range(GL // L):
            plsc.store_scatter(csrc, [cnt + t * L + iota],
                               jnp.zeros((L,), jnp.int32))
            plsc.store_scatter(cdst, [cnt + t * L + iota],
                               jnp.full((L,), NLOC, dtype=jnp.int32))

        ngroups = (cnt + GL - 1) // GL
        nouter = (ngroups + NBUF - 1) // NBUF

        for bnum in range(NBUF):
            @pl.when(bnum < ngroups)
            def _prime():
                pltpu.async_copy(bm_hbm.at[csrc.at[pl.ds(bnum * GL, GL)]],
                                 ring.at[bnum], rsems.at[bnum])

        def outer_body(o, carry):
            for bnum in range(NBUF):
                g = o * NBUF + bnum

                @pl.when(g < ngroups)
                def _drain():
                    pltpu.make_async_copy(bm_hbm.at[pl.ds(0, GL)],
                                          ring.at[bnum], rsems.at[bnum]).wait()
                    for half in range(GL // L):
                        ld16 = plsc.load_gather(
                            cdst, [g * GL + half * L + iota])
                        for j in range(L):
                            rb = ld16[j] * DD
                            rv = [ring[bnum, half * L + j, pl.ds(k * L, L)]
                                  for k in range(DD // L)]
                            av = [acc[pl.ds(rb + k * L, L)]
                                  for k in range(DD // L)]
                            mx = [jnp.maximum(av[k], rv[k])
                                  for k in range(DD // L)]
                            for k in range(DD // L):
                                acc[pl.ds(rb + k * L, L)] = mx[k]

                    @pl.when(g + NBUF < ngroups)
                    def _refire():
                        pltpu.async_copy(
                            bm_hbm.at[csrc.at[pl.ds((g + NBUF) * GL, GL)]],
                            ring.at[bnum], rsems.at[bnum])
            return carry

        lax.fori_loop(0, nouter, outer_body, 0)

    fire_chunk(0, 0)

    def chunk_loop(c, carry):
        b = c % 2

        @pl.when(b == 0)
        def _w0():
            wait_chunk(0)

        @pl.when(b == 1)
        def _w1():
            wait_chunk(1)

        @pl.when((b == 0) & (c + 1 < NCHUNK))
        def _f1():
            fire_chunk(c + 1, 1)

        @pl.when((b == 1) & (c + 1 < NCHUNK))
        def _f0():
            fire_chunk(c + 1, 0)

        process_chunk(b)
        return carry

    lax.fori_loop(0, NCHUNK, chunk_loop, 0)
    pltpu.sync_copy(acc.at[pl.ds(0, NLOC * DD)],
                    out_hbm.at[pl.ds(base * DD, NLOC * DD)])


def _segmax(edges, bm):
    mesh = plsc.VectorSubcoreMesh(core_axis_name="c", subcore_axis_name="s",
                                  num_cores=NC, num_subcores=NS)
    k = functools.partial(
        pl.kernel,
        out_type=jax.ShapeDtypeStruct((NPAD * DD,), jnp.float32),
        mesh=mesh,
        compiler_params=pltpu.CompilerParams(needs_layout_passes=False),
        scratch_types=[
            pltpu.VMEM((2, 2, EC), jnp.int32),
            pltpu.VMEM((EC + GL,), jnp.int32),
            pltpu.VMEM((EC + GL,), jnp.int32),
            pltpu.VMEM(((NLOC + 1) * DD,), jnp.float32),
            pltpu.VMEM((NBUF, GL, DD), jnp.float32),
            pltpu.SemaphoreType.DMA((2,)),
            pltpu.SemaphoreType.DMA((NBUF,)),
        ],
    )(_segmax_body)
    return k(edges, bm)


def _norm_body(sm_ref, a_ref, b_ref, batch_ref, out_ref):
    sm = sm_ref[...][:NN]
    agg = a_ref[...] + b_ref[...] + sm
    agg = jnp.where(jnp.isfinite(sm), agg, 0.0)
    batch = batch_ref[...]
    gids = lax.broadcasted_iota(jnp.int32, (1, GG), 1)
    oh = (batch == gids).astype(jnp.float32)
    cnt = jnp.maximum(jnp.sum(oh, axis=0, keepdims=True), 1.0)
    sums = lax.dot_general(oh, agg, (((0,), (0,)), ((), ())),
                           preferred_element_type=jnp.float32,
                           precision=jax.lax.Precision.HIGHEST)
    sqs = lax.dot_general(oh, agg * agg, (((0,), (0,)), ((), ())),
                          preferred_element_type=jnp.float32,
                          precision=jax.lax.Precision.HIGHEST)
    mean = sums / cnt.T
    var = jnp.maximum(sqs / cnt.T - mean * mean, 0.0)
    scale = jax.lax.rsqrt(var + 1e-5)
    mean_n = lax.dot_general(oh, mean, (((1,), (0,)), ((), ())),
                             preferred_element_type=jnp.float32,
                             precision=jax.lax.Precision.HIGHEST)
    scale_n = lax.dot_general(oh, scale, (((1,), (0,)), ((), ())),
                              preferred_element_type=jnp.float32,
                              precision=jax.lax.Precision.HIGHEST)
    v = (agg - mean_n) * scale_n
    out_ref[...] = jnp.where(v >= 0, v, 0.01 * v)


def _norm(sm, a, b, batch):
    return pl.pallas_call(
        _norm_body,
        in_specs=[
            pl.BlockSpec((NPAD, DD), lambda: (0, 0)),
            pl.BlockSpec((NN, DD), lambda: (0, 0)),
            pl.BlockSpec((1, DD), lambda: (0, 0)),
            pl.BlockSpec((NN, 1), lambda: (0, 0)),
        ],
        out_specs=pl.BlockSpec((NN, DD), lambda: (0, 0)),
        out_shape=jax.ShapeDtypeStruct((NN, DD), jnp.float32),
    )(sm, a, b.reshape(1, DD), batch.reshape(NN, 1))


def kernel(x, edge_index, batch, W, b):
    w1 = W[:, :DD]
    w2 = W[:, DD:]
    wc = jnp.concatenate([(w1 - w2).T, w2.T], axis=1)
    a, bm = _node_matmul(x, wc)

    pad = EPAD - EE
    padcol = jnp.concatenate(
        [jnp.zeros((1, pad), jnp.int32),
         jnp.full((1, pad), jnp.int32(1 << 28))], axis=0)
    edges = jnp.concatenate([edge_index, padcol], axis=1)

    smflat = _segmax(edges, bm)
    sm = smflat.reshape(NPAD, DD)
    return _norm(sm, a, b, batch)


# speedup vs baseline: 2.0397x; 2.0397x over previous
import functools

import jax
import jax.numpy as jnp
from jax import lax
from jax.experimental import pallas as pl
from jax.experimental.pallas import tpu as pltpu
from jax.experimental.pallas import tpu_sc as plsc

NN = 10000
EE = 320000
DD = 128
GG = 16

NC = 2
NS = 16
NW = NC * NS
L = 16

NLOC = 313
NPAD = NW * NLOC
EC = 8192
NCHUNK = 40
EPAD = NCHUNK * EC
NBUF = 20
GL = 16


def _matmul_body(x_ref, wc_ref, a_ref, bm_ref):
    p = jnp.dot(x_ref[...], wc_ref[...],
                preferred_element_type=jnp.float32,
                precision=jax.lax.Precision.HIGHEST)
    a_ref[...] = p[:, :DD]
    bm_ref[...] = p[:, DD:]


def _node_matmul(x, wc):
    bn = 2000
    return pl.pallas_call(
        _matmul_body,
        grid=(NN // bn,),
        in_specs=[
            pl.BlockSpec((bn, DD), lambda i: (i, 0)),
            pl.BlockSpec((DD, 2 * DD), lambda i: (0, 0)),
        ],
        out_specs=[
            pl.BlockSpec((bn, DD), lambda i: (i, 0)),
            pl.BlockSpec((bn, DD), lambda i: (i, 0)),
        ],
        out_shape=[
            jax.ShapeDtypeStruct((NN, DD), jnp.float32),
            jax.ShapeDtypeStruct((NN, DD), jnp.float32),
        ],
    )(x, wc)


def _segmax_body(edge_hbm, bm_hbm, out_hbm,
                 ebuf, csrc, cdst, acc, ring, esems, rsems):
    wid = lax.axis_index("s") * NC + lax.axis_index("c")
    base = wid * NLOC
    lower = base
    upper = base + NLOC
    iota = lax.iota(jnp.int32, L)
    neg = jnp.full((L,), -jnp.inf, dtype=jnp.float32)

    def init_body(i, carry):
        for k in range(8):
            acc[pl.ds(i * 8 * L + k * L, L)] = neg
        return carry

    lax.fori_loop(0, (NLOC + 1) * DD // (8 * L), init_body, 0)

    def fire_chunk(c, b):
        pltpu.async_copy(edge_hbm.at[:, pl.ds(c * EC, EC)], ebuf.at[b],
                         esems.at[b])

    def wait_chunk(b):
        pltpu.make_async_copy(edge_hbm.at[:, pl.ds(0, EC)], ebuf.at[b],
                              esems.at[b]).wait()

    def process_chunk(b):
        def scan_body(i, cnt):
            da = ebuf[b, 1, pl.ds(2 * i * L, L)]
            sa = ebuf[b, 0, pl.ds(2 * i * L, L)]
            db = ebuf[b, 1, pl.ds((2 * i + 1) * L, L)]
            sb = ebuf[b, 0, pl.ds((2 * i + 1) * L, L)]
            ma = (da >= lower) & (da < upper)
            mb = (db >= lower) & (db < upper)
            pca = plsc.all_reduce_population_count(ma)[0]
            pcb = plsc.all_reduce_population_count(mb)[0]
            plsc.store_compressed(csrc.at[pl.ds(cnt, L)], sa, mask=ma)
            plsc.store_compressed(cdst.at[pl.ds(cnt, L)], da - lower,
                                  mask=ma)
            cnt2 = cnt + pca
            plsc.store_compressed(csrc.at[pl.ds(cnt2, L)], sb, mask=mb)
            plsc.store_compressed(cdst.at[pl.ds(cnt2, L)], db - lower,
                                  mask=mb)
            return cnt2 + pcb

        cnt = lax.fori_loop(0, EC // L // 2, scan_body, jnp.int32(0))

        for t in range(GL // L):
            plsc.store_scatter(csrc, [cnt + t * L + iota],
                               jnp.zeros((L,), jnp.int32))
            plsc.store_scatter(cdst, [cnt + t * L + iota],
                               jnp.full((L,), NLOC, dtype=jnp.int32))

        ngroups = (cnt + GL - 1) // GL
        nouter = (ngroups + NBUF - 1) // NBUF

        for bnum in range(NBUF):
            @pl.when(bnum < ngroups)
            def _prime():
                pltpu.async_copy(bm_hbm.at[csrc.at[pl.ds(bnum * GL, GL)]],
                                 ring.at[bnum], rsems.at[bnum])

        def outer_body(o, carry):
            for bnum in range(NBUF):
                g = o * NBUF + bnum

                @pl.when(g < ngroups)
                def _drain():
                    pltpu.make_async_copy(bm_hbm.at[pl.ds(0, GL)],
                                          ring.at[bnum], rsems.at[bnum]).wait()
                    for half in range(GL // L):
                        ld16 = plsc.load_gather(
                            cdst, [g * GL + half * L + iota])
                        for j in range(L):
                            rb = ld16[j] * DD
                            rv = [ring[bnum, half * L + j, pl.ds(k * L, L)]
                                  for k in range(DD // L)]
                            av = [acc[pl.ds(rb + k * L, L)]
                                  for k in range(DD // L)]
                            mx = [jnp.maximum(av[k], rv[k])
                                  for k in range(DD // L)]
                            for k in range(DD // L):
                                acc[pl.ds(rb + k * L, L)] = mx[k]

                    @pl.when(g + NBUF < ngroups)
                    def _refire():
                        pltpu.async_copy(
                            bm_hbm.at[csrc.at[pl.ds((g + NBUF) * GL, GL)]],
                            ring.at[bnum], rsems.at[bnum])
            return carry

        lax.fori_loop(0, nouter, outer_body, 0)

    fire_chunk(0, 0)

    def chunk_loop(c, carry):
        b = c % 2

        @pl.when(b == 0)
        def _w0():
            wait_chunk(0)

        @pl.when(b == 1)
        def _w1():
            wait_chunk(1)

        @pl.when((b == 0) & (c + 1 < NCHUNK))
        def _f1():
            fire_chunk(c + 1, 1)

        @pl.when((b == 1) & (c + 1 < NCHUNK))
        def _f0():
            fire_chunk(c + 1, 0)

        process_chunk(b)
        return carry

    lax.fori_loop(0, NCHUNK, chunk_loop, 0)
    pltpu.sync_copy(acc.at[pl.ds(0, NLOC * DD)],
                    out_hbm.at[pl.ds(base * DD, NLOC * DD)])


def _segmax(edges, bm):
    mesh = plsc.VectorSubcoreMesh(core_axis_name="c", subcore_axis_name="s",
                                  num_cores=NC, num_subcores=NS)
    k = functools.partial(
        pl.kernel,
        out_type=jax.ShapeDtypeStruct((NPAD * DD,), jnp.float32),
        mesh=mesh,
        compiler_params=pltpu.CompilerParams(needs_layout_passes=False),
        scratch_types=[
            pltpu.VMEM((2, 2, EC), jnp.int32),
            pltpu.VMEM((EC + GL,), jnp.int32),
            pltpu.VMEM((EC + GL,), jnp.int32),
            pltpu.VMEM(((NLOC + 1) * DD,), jnp.float32),
            pltpu.VMEM((NBUF, GL, DD), jnp.float32),
            pltpu.SemaphoreType.DMA((2,)),
            pltpu.SemaphoreType.DMA((NBUF,)),
        ],
    )(_segmax_body)
    return k(edges, bm)


def _norm_body(sm_ref, a_ref, b_ref, batch_ref, out_ref):
    sm = sm_ref[...][:NN]
    agg = a_ref[...] + b_ref[...] + sm
    agg = jnp.where(jnp.isfinite(sm), agg, 0.0)
    batch = batch_ref[...]
    gids = lax.broadcasted_iota(jnp.int32, (1, GG), 1)
    oh = (batch == gids).astype(jnp.float32)
    cnt = jnp.maximum(jnp.sum(oh, axis=0, keepdims=True), 1.0)
    sums = lax.dot_general(oh, agg, (((0,), (0,)), ((), ())),
                           preferred_element_type=jnp.float32,
                           precision=jax.lax.Precision.HIGHEST)
    sqs = lax.dot_general(oh, agg * agg, (((0,), (0,)), ((), ())),
                          preferred_element_type=jnp.float32,
                          precision=jax.lax.Precision.HIGHEST)
    mean = sums / cnt.T
    var = jnp.maximum(sqs / cnt.T - mean * mean, 0.0)
    scale = jax.lax.rsqrt(var + 1e-5)
    mean_n = lax.dot_general(oh, mean, (((1,), (0,)), ((), ())),
                             preferred_element_type=jnp.float32,
                             precision=jax.lax.Precision.HIGHEST)
    scale_n = lax.dot_general(oh, scale, (((1,), (0,)), ((), ())),
                              preferred_element_type=jnp.float32,
                              precision=jax.lax.Precision.HIGHEST)
    v = (agg - mean_n) * scale_n
    out_ref[...] = jnp.where(v >= 0, v, 0.01 * v)


def _norm(sm, a, b, batch):
    return pl.pallas_call(
        _norm_body,
        in_specs=[
            pl.BlockSpec((NPAD, DD), lambda: (0, 0)),
            pl.BlockSpec((NN, DD), lambda: (0, 0)),
            pl.BlockSpec((1, DD), lambda: (0, 0)),
            pl.BlockSpec((NN, 1), lambda: (0, 0)),
        ],
        out_specs=pl.BlockSpec((NN, DD), lambda: (0, 0)),
        out_shape=jax.ShapeDtypeStruct((NN, DD), jnp.float32),
    )(sm, a, b.reshape(1, DD), batch.reshape(NN, 1))


def kernel(x, edge_index, batch, W, b):
    w1 = W[:, :DD]
    w2 = W[:, DD:]
    wc = jnp.concatenate([(w1 - w2).T, w2.T], axis=1)
    a, bm = _node_matmul(x, wc)

    pad = EPAD - EE
    padcol = jnp.concatenate(
        [jnp.zeros((1, pad), jnp.int32),
         jnp.full((1, pad), jnp.int32(1 << 28))], axis=0)
    edges = jnp.concatenate([edge_index, padcol], axis=1)

    smflat = _segmax(edges, bm)
    sm = smflat.reshape(NPAD, DD)
    return _norm(sm, a, b, batch)
